# baseline (speedup 1.0000x reference)
import functools

import jax
import jax.numpy as jnp
from jax import lax
from jax.experimental import pallas as pl
from jax.experimental.pallas import tpu as pltpu
from jax.experimental.pallas import tpu_sc as plsc

NUM_DOCS = 5000
NW = 32
CHUNK = 128
N_TILES = 16
WP = 64
NP1 = 4

_UNPACK_EVEN_ODD = True

_MESH = plsc.VectorSubcoreMesh(core_axis_name="c", subcore_axis_name="s")
_SC_PARAMS = pltpu.CompilerParams(
    needs_layout_passes=False, use_tc_tiling_on_sc=False
)


def _panel_perm(width):
    if not _UNPACK_EVEN_ODD:
        return list(range(width))
    pi = [0] * width
    for j in range(width // 32):
        for m in range(16):
            pi[32 * j + 2 * m] = 32 * j + m
            pi[32 * j + 2 * m + 1] = 32 * j + 16 + m
    return pi


def _linear_body(xr, wr, br, outr):
    outr[...] = (
        jnp.dot(xr[...], wr[...], preferred_element_type=jnp.float32) + br[...]
    )


def _word_linear(word, WlinT, b_lin):
    M, K = word.shape
    Nf = WlinT.shape[1]
    BM = 1000
    return pl.pallas_call(
        _linear_body,
        grid=(M // BM,),
        in_specs=[
            pl.BlockSpec((BM, K), lambda i: (i, 0)),
            pl.BlockSpec((K, Nf), lambda i: (0, 0)),
            pl.BlockSpec((1, Nf), lambda i: (0, 0)),
        ],
        out_specs=pl.BlockSpec((BM, Nf), lambda i: (i, 0)),
        out_shape=jax.ShapeDtypeStruct((M, Nf), jnp.float32),
    )(word, WlinT, b_lin.reshape(1, -1))


def _dinv_body(degr, outr):
    n = outr.shape[0]
    d = degr[0, :n] + degr[1, :n]
    di = jnp.where(d > 0, lax.rsqrt(jnp.where(d > 0, d, 1.0)), 0.0)
    outr[...] = jnp.broadcast_to(di[:, None], outr.shape)


def _dinv_tc(degp, N):
    return pl.pallas_call(
        _dinv_body,
        in_specs=[pl.BlockSpec(degp.shape, lambda: (0, 0))],
        out_specs=pl.BlockSpec((N, 8), lambda: (0, 0)),
        out_shape=jax.ShapeDtypeStruct((N, 8), jnp.float32),
    )(degp)


def _conv1_mm_body(h0r, w1r, dvr, gsr):
    g1 = jnp.dot(h0r[...], w1r[0], preferred_element_type=jnp.float32)
    gsr[0] = (g1 * dvr[:, 0:1]).astype(jnp.bfloat16)


def _conv1_mm(h0, W1s, dinv):
    M, K = h0.shape
    BM = 2000
    return pl.pallas_call(
        _conv1_mm_body,
        grid=(NP1, M // BM),
        in_specs=[
            pl.BlockSpec((BM, K), lambda p, i: (i, 0)),
            pl.BlockSpec((1, K, WP), lambda p, i: (p, 0, 0)),
            pl.BlockSpec((BM, 8), lambda p, i: (i, 0)),
        ],
        out_specs=pl.BlockSpec((1, BM, WP), lambda p, i: (p, i, 0)),
        out_shape=jax.ShapeDtypeStruct((NP1, M, WP), jnp.bfloat16),
    )(h0, W1s, dinv)


def _conv2_mm_body(pr, dvr, b1r, w2r, gs2r):
    dinv = dvr[:, 0:1]
    g2 = jnp.zeros(gs2r.shape, jnp.float32)
    for p in range(NP1):
        agg = pr[0, p] + pr[1, p]
        h1 = jnp.maximum(agg * dinv + b1r[p], 0.0)
        g2 = g2 + jnp.dot(h1, w2r[p], preferred_element_type=jnp.float32)
    gs2r[...] = g2 * dinv


def _conv2_mm(P1, dinv, b1s, W2s):
    M = P1.shape[2]
    D2 = W2s.shape[2]
    BM = 1000
    return pl.pallas_call(
        _conv2_mm_body,
        grid=(M // BM,),
        in_specs=[
            pl.BlockSpec((2, NP1, BM, WP), lambda i: (0, 0, i, 0)),
            pl.BlockSpec((BM, 8), lambda i: (i, 0)),
            pl.BlockSpec((NP1, 1, WP), lambda i: (0, 0, 0)),
            pl.BlockSpec((NP1, WP, D2), lambda i: (0, 0, 0)),
        ],
        out_specs=pl.BlockSpec((BM, D2), lambda i: (i, 0)),
        out_shape=jax.ShapeDtypeStruct((M, D2), jnp.float32),
    )(P1, dinv, b1s, W2s)


def _final_body(pr, dvr, b2r, outr):
    agg = pr[0, 0] + pr[1, 0]
    outr[...] = agg * dvr[:, 0:1] + b2r[...]


def _final(P2, dinv, b2p):
    M, D2 = P2.shape[2], P2.shape[3]
    BM = 1000
    return pl.pallas_call(
        _final_body,
        grid=(M // BM,),
        in_specs=[
            pl.BlockSpec((2, 1, BM, D2), lambda i: (0, 0, i, 0)),
            pl.BlockSpec((BM, 8), lambda i: (i, 0)),
            pl.BlockSpec((1, D2), lambda i: (0, 0)),
        ],
        out_specs=pl.BlockSpec((BM, D2), lambda i: (i, 0)),
        out_shape=jax.ShapeDtypeStruct((M, D2), jnp.float32),
    )(P2, dinv, b2p.reshape(1, -1))


def _make_deg_kernel(NG, PW, DEGP):
    stripe = DEGP // N_TILES
    assert stripe % 128 == 0 and NG % 8 == 0

    @functools.partial(
        pl.kernel,
        out_type=jax.ShapeDtypeStruct((2 * DEGP,), jnp.float32),
        mesh=_MESH,
        compiler_params=_SC_PARAMS,
        scratch_types=[
            pltpu.VMEM_SHARED((DEGP,), jnp.float32),
            pltpu.VMEM((PW,), jnp.float32),
            pltpu.VMEM((NG, CHUNK), jnp.int32),
            pltpu.VMEM((stripe,), jnp.float32),
            pltpu.SemaphoreType.DMA,
        ],
    )
    def deg_kernel(dst_hbm, ew_hbm, out_hbm, acc, ewv, dstv, zv, sem):
        c = lax.axis_index("c")
        s = lax.axis_index("s")
        w = s * 2 + c
        base = pl.multiple_of(s * stripe, 128)

        def zero_body(i, _):
            zv[pl.ds(i * 16, 16)] = jnp.zeros((16,), jnp.float32)
            return 0

        lax.fori_loop(0, stripe // 16, zero_body, 0)
        pltpu.sync_copy(zv, acc.at[pl.ds(base, stripe)])
        plsc.subcore_barrier()

        pltpu.sync_copy(ew_hbm.at[pl.ds(w * PW, PW)], ewv)
        pltpu.sync_copy(dst_hbm.at[w], dstv)

        def scat(g0, _):
            descs = []
            for k in range(8):
                g = g0 * 8 + k
                descs.append(
                    pltpu.async_copy(
                        ewv.at[pl.ds(g * CHUNK, CHUNK)],
                        acc.at[dstv.at[g]],
                        sem,
                        add=True,
                    )
                )
            for d in descs:
                d.wait()
            return 0

        lax.fori_loop(0, NG // 8, scat, 0)
        plsc.subcore_barrier()
        pltpu.sync_copy(
            acc.at[pl.ds(base, stripe)],
            out_hbm.at[pl.ds(pl.multiple_of(c * DEGP + base, 128), stripe)],
        )

    return deg_kernel


def _make_agg_kernel(NROWS, D, NPASS, NG, is_bf16):
    stripe = 624
    last = NROWS - 15 * stripe
    assert NROWS == 10000 and NG % 4 == 0 and NG >= 12
    tdtype = jnp.bfloat16 if is_bf16 else jnp.float32

    @functools.partial(
        pl.kernel,
        out_type=jax.ShapeDtypeStruct((2, NPASS, NROWS, D), jnp.float32),
        mesh=_MESH,
        compiler_params=_SC_PARAMS,
        scratch_types=[
            pltpu.VMEM_SHARED((NROWS, D), jnp.float32),
            pltpu.VMEM_SHARED((NROWS, D), tdtype),
            [pltpu.VMEM((3, CHUNK), jnp.int32) for _ in range(4)],
            [pltpu.VMEM((CHUNK, D), tdtype) for _ in range(4)],
            [pltpu.VMEM((CHUNK, D), jnp.float32) for _ in range(2)],
            [pltpu.VMEM((CHUNK,), jnp.int32) for _ in range(2)],
            [pltpu.SemaphoreType.DMA for _ in range(4)],
            [pltpu.SemaphoreType.DMA for _ in range(4)],
            [pltpu.SemaphoreType.DMA for _ in range(2)],
        ],
    )
    def agg_kernel(g_hbm, edges_hbm, out_hbm,
                   acc, tspm, ering, gbufs, sbufs, sidx, esems, gsems, ssems):
        c = lax.axis_index("c")
        s = lax.axis_index("s")
        w = s * 2 + c

        def row_chunks(emit):
            base = pl.multiple_of(s * stripe, 16)

            @pl.when(s < 15)
            def _():
                for j in range(stripe // CHUNK):
                    emit(pl.multiple_of(base + j * CHUNK, 16), CHUNK)
                r = stripe - (stripe // CHUNK) * CHUNK
                if r:
                    emit(pl.multiple_of(base + stripe - r, 16), r)

            @pl.when(s == 15)
            def _():
                for j in range(last // CHUNK):
                    emit(pl.multiple_of(base + j * CHUNK, 16), CHUNK)
                r = last - (last // CHUNK) * CHUNK
                if r:
                    emit(pl.multiple_of(base + last - r, 16), r)

        def zero_sbuf0(i, _):
            sbufs[0][i // (D // 16), pl.ds((i % (D // 16)) * 16, 16)] = (
                jnp.zeros((16,), jnp.float32)
            )
            return 0

        def zero_acc():
            lax.fori_loop(0, CHUNK * (D // 16), zero_sbuf0, 0)
            row_chunks(
                lambda r0, n: pltpu.sync_copy(
                    sbufs[0].at[pl.ds(0, n)], acc.at[pl.ds(r0, n)]
                )
            )
            plsc.subcore_barrier()

        for p in range(NPASS):
            row_chunks(
                lambda r0, n: pltpu.sync_copy(
                    g_hbm.at[p].at[pl.ds(r0, n)], tspm.at[pl.ds(r0, n)]
                )
            )
            zero_acc()
            table = tspm

            def fire_estage(t, k):
                pltpu.async_copy(edges_hbm.at[w].at[t], ering[k], esems[k])

            def wait_estage(t, k):
                pltpu.make_async_copy(
                    edges_hbm.at[w].at[t], ering[k], esems[k]
                ).wait()

            def fire_gather(t, k):
                pltpu.async_copy(table.at[ering[k].at[0]], gbufs[k], gsems[k])

            def wait_gather(t, k):
                pltpu.make_async_copy(
                    table.at[ering[k].at[0]], gbufs[k], gsems[k]
                ).wait()

            def fire_scatter(k):
                pltpu.async_copy(
                    sbufs[k], acc.at[sidx[k]], ssems[k], add=True
                )

            def wait_scatter(k):
                pltpu.make_async_copy(
                    sbufs[k], acc.at[sidx[k]], ssems[k]
                ).wait()

            def compute(t, ke, ks):
                gbuf = gbufs[ke]
                sbuf = sbufs[ks]

                for q in range(CHUNK // 16):
                    sidx[ks][pl.ds(q * 16, 16)] = ering[ke][
                        1, pl.ds(q * 16, 16)
                    ]

                @plsc.parallel_loop(0, CHUNK, unroll=4)
                def body(b):
                    ew = plsc.bitcast(
                        plsc.load_gather(
                            ering[ke],
                            [
                                jnp.full((16,), 2, jnp.int32),
                                jnp.full((16,), b, jnp.int32),
                            ],
                        ),
                        jnp.float32,
                    )
                    if is_bf16:
                        for j in range(D // 32):
                            v = gbuf[b, pl.ds(j * 32, 32)]
                            va, vb = plsc.unpack(
                                v,
                                format=plsc.PackFormat.INTERLEAVED,
                                preferred_element_type=jnp.float32,
                            )
                            sbuf[b, pl.ds(j * 32, 16)] = va * ew
                            sbuf[b, pl.ds(j * 32 + 16, 16)] = vb * ew
                    else:
                        for j in range(D // 16):
                            sbuf[b, pl.ds(j * 16, 16)] = (
                                gbuf[b, pl.ds(j * 16, 16)] * ew
                            )

            def slot(t, ph, first=False, fire_e=True, fire_g=True):
                if fire_e:
                    fire_estage(t + 3, (ph + 3) % 4)
                if fire_g:
                    wait_estage(t + 2, (ph + 2) % 4)
                if not first:
                    wait_scatter(ph % 2)
                if fire_g:
                    fire_gather(t + 2, (ph + 2) % 4)
                wait_gather(t, ph)
                compute(t, ph, ph % 2)
                fire_scatter(ph % 2)

            fire_estage(0, 0)
            fire_estage(1, 1)
            fire_estage(2, 2)
            wait_estage(0, 0)
            fire_gather(0, 0)
            wait_estage(1, 1)
            fire_gather(1, 1)
            slot(0, 0, first=True)
            slot(1, 1, first=True)

            def slot_group(i, _):
                t0 = 2 + i * 4
                for j in range(4):
                    slot(t0 + j, (2 + j) % 4)
                return 0

            lax.fori_loop(0, (NG - 8) // 4, slot_group, 0)

            for t in range(NG - 6, NG):
                slot(t, t % 4, fire_e=(t + 3 < NG), fire_g=(t + 2 < NG))
            wait_scatter((NG - 2) % 2)
            wait_scatter((NG - 1) % 2)
            plsc.subcore_barrier()

            row_chunks(
                lambda r0, n: pltpu.sync_copy(
                    acc.at[pl.ds(r0, n)], out_hbm.at[c, p, pl.ds(r0, n)]
                )
            )
            if p + 1 < NPASS:
                plsc.subcore_barrier()

    return agg_kernel


def kernel(x, edge_index, edge_attr, num_docs, W_lin, b_lin, W1, b1, W2, b2):
    N = x.shape[0]
    E = edge_index.shape[1]

    doc_feats = lax.dynamic_slice_in_dim(x, num_docs - NUM_DOCS, NUM_DOCS, axis=0)
    word_feats = lax.dynamic_slice_in_dim(x, num_docs, N - NUM_DOCS, axis=0)
    word_feats = word_feats[:, : W_lin.shape[1]]

    wout = _word_linear(word_feats, W_lin.T, b_lin)
    h0 = jnp.concatenate([doc_feats, wout], axis=0)

    loop = jnp.arange(N, dtype=edge_index.dtype)
    srcA = jnp.concatenate([edge_index[0], loop])
    dstA = jnp.concatenate([edge_index[1], loop])
    ewA = jnp.concatenate([edge_attr, jnp.ones((N,), edge_attr.dtype)])
    ET = E + N
    NG = max(16, ((-(-ET // (NW * CHUNK)) + 7) // 8) * 8)
    PW = NG * CHUNK
    EP = NW * PW
    pad = EP - ET
    src = jnp.concatenate([srcA, jnp.zeros((pad,), edge_index.dtype)])
    dst = jnp.concatenate([dstA, jnp.zeros((pad,), edge_index.dtype)])
    ew = jnp.concatenate([ewA, jnp.zeros((pad,), edge_attr.dtype)])
    src3 = src.reshape(NW, NG, CHUNK)
    dst3 = dst.reshape(NW, NG, CHUNK)
    ew3 = ew.reshape(NW, NG, CHUNK)
    edgesP = jnp.stack(
        [src3, dst3, lax.bitcast_convert_type(ew3, jnp.int32)], axis=2
    )

    DEGP = ((N + 2047) // 2048) * 2048
    degp = _make_deg_kernel(NG, PW, DEGP)(dst3, ew).reshape(2, DEGP)
    dinv = _dinv_tc(degp, N)

    FW = NP1 * WP
    H1 = W1.shape[1]
    W1p = jnp.pad(W1, ((0, 0), (0, FW - H1)))
    W1perm = W1p[:, jnp.array(_panel_perm(FW))]
    W1s = jnp.stack([W1perm[:, p * WP:(p + 1) * WP] for p in range(NP1)])
    b1p = jnp.pad(b1, (0, FW - H1))
    b1s = jnp.stack([b1p[p * WP:(p + 1) * WP] for p in range(NP1)]).reshape(
        NP1, 1, WP
    )
    gss = _conv1_mm(h0, W1s, dinv)

    P1 = _make_agg_kernel(N, WP, NP1, NG, True)(gss, edgesP)

    D2 = ((W2.shape[1] + 15) // 16) * 16
    W2p = jnp.pad(W2, ((0, FW - W2.shape[0]), (0, D2 - W2.shape[1])))
    W2s = jnp.stack([W2p[p * WP:(p + 1) * WP] for p in range(NP1)])
    b2p = jnp.pad(b2, (0, D2 - b2.shape[0]))
    gs2 = _conv2_mm(P1, dinv, b1s, W2s)

    P2 = _make_agg_kernel(N, D2, 1, NG, False)(
        gs2.reshape(1, N, D2), edgesP
    )

    out16 = _final(P2, dinv, b2p)
    return out16[:, : W2.shape[1]]


# speedup vs baseline: 1.0392x; 1.0392x over previous
import functools

import jax
import jax.numpy as jnp
from jax import lax
from jax.experimental import pallas as pl
from jax.experimental.pallas import tpu as pltpu
from jax.experimental.pallas import tpu_sc as plsc

NUM_DOCS = 5000
NW = 32
CHUNK = 128
N_TILES = 16
WP = 64
NP1 = 4

_UNPACK_EVEN_ODD = True

_MESH = plsc.VectorSubcoreMesh(core_axis_name="c", subcore_axis_name="s")
_SC_PARAMS = pltpu.CompilerParams(
    needs_layout_passes=False, use_tc_tiling_on_sc=False
)


def _panel_perm(width):
    if not _UNPACK_EVEN_ODD:
        return list(range(width))
    pi = [0] * width
    for j in range(width // 32):
        for m in range(16):
            pi[32 * j + 2 * m] = 32 * j + m
            pi[32 * j + 2 * m + 1] = 32 * j + 16 + m
    return pi


def _linear_body(xr, wr, br, outr):
    outr[...] = (
        jnp.dot(xr[...], wr[...], preferred_element_type=jnp.float32) + br[...]
    )


def _word_linear(word, WlinT, b_lin):
    M, K = word.shape
    Nf = WlinT.shape[1]
    BM = 1000
    return pl.pallas_call(
        _linear_body,
        grid=(M // BM,),
        in_specs=[
            pl.BlockSpec((BM, K), lambda i: (i, 0)),
            pl.BlockSpec((K, Nf), lambda i: (0, 0)),
            pl.BlockSpec((1, Nf), lambda i: (0, 0)),
        ],
        out_specs=pl.BlockSpec((BM, Nf), lambda i: (i, 0)),
        out_shape=jax.ShapeDtypeStruct((M, Nf), jnp.float32),
    )(word, WlinT, b_lin.reshape(1, -1))


def _word_dinv_body(xr, wr, br, degr, outr, dvr):
    outr[...] = (
        jnp.dot(xr[...], wr[...], preferred_element_type=jnp.float32) + br[...]
    )

    @pl.when(pl.program_id(0) == 0)
    def _():
        n = dvr.shape[0]
        d = degr[0, :n] + degr[1, :n]
        di = jnp.where(d > 0, lax.rsqrt(jnp.where(d > 0, d, 1.0)), 0.0)
        dvr[...] = jnp.broadcast_to(di[:, None], dvr.shape)


def _word_dinv(word, WlinT, b_lin, degp, N):
    M, K = word.shape
    Nf = WlinT.shape[1]
    BM = 1000
    return pl.pallas_call(
        _word_dinv_body,
        grid=(M // BM,),
        in_specs=[
            pl.BlockSpec((BM, K), lambda i: (i, 0)),
            pl.BlockSpec((K, Nf), lambda i: (0, 0)),
            pl.BlockSpec((1, Nf), lambda i: (0, 0)),
            pl.BlockSpec(degp.shape, lambda i: (0, 0)),
        ],
        out_specs=[
            pl.BlockSpec((BM, Nf), lambda i: (i, 0)),
            pl.BlockSpec((N, 8), lambda i: (0, 0)),
        ],
        out_shape=[
            jax.ShapeDtypeStruct((M, Nf), jnp.float32),
            jax.ShapeDtypeStruct((N, 8), jnp.float32),
        ],
    )(word, WlinT, b_lin.reshape(1, -1), degp)


def _conv1_mm_body(h0r, w1r, dvr, gsr):
    g1 = jnp.dot(h0r[...], w1r[0], preferred_element_type=jnp.float32)
    gsr[0] = (g1 * dvr[:, 0:1]).astype(jnp.bfloat16)


def _conv1_mm(h0, W1s, dinv):
    M, K = h0.shape
    BM = 2000
    return pl.pallas_call(
        _conv1_mm_body,
        grid=(NP1, M // BM),
        in_specs=[
            pl.BlockSpec((BM, K), lambda p, i: (i, 0)),
            pl.BlockSpec((1, K, WP), lambda p, i: (p, 0, 0)),
            pl.BlockSpec((BM, 8), lambda p, i: (i, 0)),
        ],
        out_specs=pl.BlockSpec((1, BM, WP), lambda p, i: (p, i, 0)),
        out_shape=jax.ShapeDtypeStruct((NP1, M, WP), jnp.bfloat16),
    )(h0, W1s, dinv)


def _conv2_mm_body(pr, dvr, b1r, w2r, gs2r):
    dinv = dvr[:, 0:1]
    g2 = jnp.zeros(gs2r.shape, jnp.float32)
    for p in range(NP1):
        agg = pr[0, p] + pr[1, p]
        h1 = jnp.maximum(agg * dinv + b1r[p], 0.0)
        g2 = g2 + jnp.dot(h1, w2r[p], preferred_element_type=jnp.float32)
    gs2r[...] = g2 * dinv


def _conv2_mm(P1, dinv, b1s, W2s):
    M = P1.shape[2]
    D2 = W2s.shape[2]
    BM = 1000
    return pl.pallas_call(
        _conv2_mm_body,
        grid=(M // BM,),
        in_specs=[
            pl.BlockSpec((2, NP1, BM, WP), lambda i: (0, 0, i, 0)),
            pl.BlockSpec((BM, 8), lambda i: (i, 0)),
            pl.BlockSpec((NP1, 1, WP), lambda i: (0, 0, 0)),
            pl.BlockSpec((NP1, WP, D2), lambda i: (0, 0, 0)),
        ],
        out_specs=pl.BlockSpec((BM, D2), lambda i: (i, 0)),
        out_shape=jax.ShapeDtypeStruct((M, D2), jnp.float32),
    )(P1, dinv, b1s, W2s)


def _final_body(pr, dvr, b2r, outr):
    agg = pr[0, 0] + pr[1, 0]
    outr[...] = agg * dvr[:, 0:1] + b2r[...]


def _final(P2, dinv, b2p):
    M, D2 = P2.shape[2], P2.shape[3]
    BM = 1000
    return pl.pallas_call(
        _final_body,
        grid=(M // BM,),
        in_specs=[
            pl.BlockSpec((2, 1, BM, D2), lambda i: (0, 0, i, 0)),
            pl.BlockSpec((BM, 8), lambda i: (i, 0)),
            pl.BlockSpec((1, D2), lambda i: (0, 0)),
        ],
        out_specs=pl.BlockSpec((BM, D2), lambda i: (i, 0)),
        out_shape=jax.ShapeDtypeStruct((M, D2), jnp.float32),
    )(P2, dinv, b2p.reshape(1, -1))


def _make_deg_kernel(NG, PW, DEGP):
    stripe = DEGP // N_TILES
    assert stripe % 128 == 0 and NG % 8 == 0

    @functools.partial(
        pl.kernel,
        out_type=jax.ShapeDtypeStruct((2 * DEGP,), jnp.float32),
        mesh=_MESH,
        compiler_params=_SC_PARAMS,
        scratch_types=[
            pltpu.VMEM_SHARED((DEGP,), jnp.float32),
            pltpu.VMEM((PW,), jnp.float32),
            pltpu.VMEM((NG, CHUNK), jnp.int32),
            pltpu.VMEM((stripe,), jnp.float32),
            pltpu.SemaphoreType.DMA,
        ],
    )
    def deg_kernel(dst_hbm, ew_hbm, out_hbm, acc, ewv, dstv, zv, sem):
        c = lax.axis_index("c")
        s = lax.axis_index("s")
        w = s * 2 + c
        base = pl.multiple_of(s * stripe, 128)

        def zero_body(i, _):
            zv[pl.ds(i * 16, 16)] = jnp.zeros((16,), jnp.float32)
            return 0

        lax.fori_loop(0, stripe // 16, zero_body, 0)
        pltpu.sync_copy(zv, acc.at[pl.ds(base, stripe)])
        plsc.subcore_barrier()

        pltpu.sync_copy(ew_hbm.at[pl.ds(w * PW, PW)], ewv)
        pltpu.sync_copy(dst_hbm.at[w], dstv)

        def scat(g0, _):
            descs = []
            for k in range(8):
                g = g0 * 8 + k
                descs.append(
                    pltpu.async_copy(
                        ewv.at[pl.ds(g * CHUNK, CHUNK)],
                        acc.at[dstv.at[g]],
                        sem,
                        add=True,
                    )
                )
            for d in descs:
                d.wait()
            return 0

        lax.fori_loop(0, NG // 8, scat, 0)
        plsc.subcore_barrier()
        pltpu.sync_copy(
            acc.at[pl.ds(base, stripe)],
            out_hbm.at[pl.ds(pl.multiple_of(c * DEGP + base, 128), stripe)],
        )

    return deg_kernel


def _make_agg_kernel(NROWS, D, NPASS, NG, is_bf16):
    stripe = 624
    last = NROWS - 15 * stripe
    assert NROWS == 10000 and NG % 4 == 0 and NG >= 12
    tdtype = jnp.bfloat16 if is_bf16 else jnp.float32

    @functools.partial(
        pl.kernel,
        out_type=jax.ShapeDtypeStruct((2, NPASS, NROWS, D), jnp.float32),
        mesh=_MESH,
        compiler_params=_SC_PARAMS,
        scratch_types=[
            pltpu.VMEM_SHARED((NROWS, D), jnp.float32),
            pltpu.VMEM_SHARED((NROWS, D), tdtype),
            [pltpu.VMEM((3, CHUNK), jnp.int32) for _ in range(4)],
            [pltpu.VMEM((CHUNK, D), tdtype) for _ in range(4)],
            [pltpu.VMEM((CHUNK, D), jnp.float32) for _ in range(2)],
            [pltpu.VMEM((CHUNK,), jnp.int32) for _ in range(2)],
            [pltpu.SemaphoreType.DMA for _ in range(4)],
            [pltpu.SemaphoreType.DMA for _ in range(4)],
            [pltpu.SemaphoreType.DMA for _ in range(2)],
        ],
    )
    def agg_kernel(g_hbm, edges_hbm, out_hbm,
                   acc, tspm, ering, gbufs, sbufs, sidx, esems, gsems, ssems):
        c = lax.axis_index("c")
        s = lax.axis_index("s")
        w = s * 2 + c

        def row_chunks(emit):
            base = pl.multiple_of(s * stripe, 16)

            @pl.when(s < 15)
            def _():
                for j in range(stripe // CHUNK):
                    emit(pl.multiple_of(base + j * CHUNK, 16), CHUNK)
                r = stripe - (stripe // CHUNK) * CHUNK
                if r:
                    emit(pl.multiple_of(base + stripe - r, 16), r)

            @pl.when(s == 15)
            def _():
                for j in range(last // CHUNK):
                    emit(pl.multiple_of(base + j * CHUNK, 16), CHUNK)
                r = last - (last // CHUNK) * CHUNK
                if r:
                    emit(pl.multiple_of(base + last - r, 16), r)

        def zero_sbuf0(i, _):
            sbufs[0][i // (D // 16), pl.ds((i % (D // 16)) * 16, 16)] = (
                jnp.zeros((16,), jnp.float32)
            )
            return 0

        def zero_acc():
            lax.fori_loop(0, CHUNK * (D // 16), zero_sbuf0, 0)
            row_chunks(
                lambda r0, n: pltpu.sync_copy(
                    sbufs[0].at[pl.ds(0, n)], acc.at[pl.ds(r0, n)]
                )
            )
            plsc.subcore_barrier()

        for p in range(NPASS):
            row_chunks(
                lambda r0, n: pltpu.sync_copy(
                    g_hbm.at[p].at[pl.ds(r0, n)], tspm.at[pl.ds(r0, n)]
                )
            )
            zero_acc()
            table = tspm

            def fire_estage(t, k):
                pltpu.async_copy(edges_hbm.at[w].at[t], ering[k], esems[k])

            def wait_estage(t, k):
                pltpu.make_async_copy(
                    edges_hbm.at[w].at[t], ering[k], esems[k]
                ).wait()

            def fire_gather(t, k):
                pltpu.async_copy(table.at[ering[k].at[0]], gbufs[k], gsems[k])

            def wait_gather(t, k):
                pltpu.make_async_copy(
                    table.at[ering[k].at[0]], gbufs[k], gsems[k]
                ).wait()

            def fire_scatter(k):
                pltpu.async_copy(
                    sbufs[k], acc.at[sidx[k]], ssems[k], add=True
                )

            def wait_scatter(k):
                pltpu.make_async_copy(
                    sbufs[k], acc.at[sidx[k]], ssems[k]
                ).wait()

            def compute(t, ke, ks):
                gbuf = gbufs[ke]
                sbuf = sbufs[ks]

                for q in range(CHUNK // 16):
                    sidx[ks][pl.ds(q * 16, 16)] = ering[ke][
                        1, pl.ds(q * 16, 16)
                    ]

                @plsc.parallel_loop(0, CHUNK, unroll=2)
                def body(b):
                    ew = plsc.bitcast(
                        plsc.load_gather(
                            ering[ke],
                            [
                                jnp.full((16,), 2, jnp.int32),
                                jnp.full((16,), b, jnp.int32),
                            ],
                        ),
                        jnp.float32,
                    )
                    if is_bf16:
                        for j in range(D // 32):
                            v = gbuf[b, pl.ds(j * 32, 32)]
                            va, vb = plsc.unpack(
                                v,
                                format=plsc.PackFormat.INTERLEAVED,
                                preferred_element_type=jnp.float32,
                            )
                            sbuf[b, pl.ds(j * 32, 16)] = va * ew
                            sbuf[b, pl.ds(j * 32 + 16, 16)] = vb * ew
                    else:
                        for j in range(D // 16):
                            sbuf[b, pl.ds(j * 16, 16)] = (
                                gbuf[b, pl.ds(j * 16, 16)] * ew
                            )

            def slot(t, ph, first=False, fire_e=True, fire_g=True):
                if fire_e:
                    fire_estage(t + 3, (ph + 3) % 4)
                if fire_g:
                    wait_estage(t + 2, (ph + 2) % 4)
                if not first:
                    wait_scatter(ph % 2)
                if fire_g:
                    fire_gather(t + 2, (ph + 2) % 4)
                wait_gather(t, ph)
                compute(t, ph, ph % 2)
                fire_scatter(ph % 2)

            fire_estage(0, 0)
            fire_estage(1, 1)
            fire_estage(2, 2)
            wait_estage(0, 0)
            fire_gather(0, 0)
            wait_estage(1, 1)
            fire_gather(1, 1)
            slot(0, 0, first=True)
            slot(1, 1, first=True)

            def slot_group(i, _):
                t0 = 2 + i * 4
                for j in range(4):
                    slot(t0 + j, (2 + j) % 4)
                return 0

            lax.fori_loop(0, (NG - 8) // 4, slot_group, 0)

            for t in range(NG - 6, NG):
                slot(t, t % 4, fire_e=(t + 3 < NG), fire_g=(t + 2 < NG))
            wait_scatter((NG - 2) % 2)
            wait_scatter((NG - 1) % 2)
            plsc.subcore_barrier()

            row_chunks(
                lambda r0, n: pltpu.sync_copy(
                    acc.at[pl.ds(r0, n)], out_hbm.at[c, p, pl.ds(r0, n)]
                )
            )
            if p + 1 < NPASS:
                plsc.subcore_barrier()

    return agg_kernel


def kernel(x, edge_index, edge_attr, num_docs, W_lin, b_lin, W1, b1, W2, b2):
    N = x.shape[0]
    E = edge_index.shape[1]

    doc_feats = lax.dynamic_slice_in_dim(x, num_docs - NUM_DOCS, NUM_DOCS, axis=0)
    word_feats = lax.dynamic_slice_in_dim(x, num_docs, N - NUM_DOCS, axis=0)
    word_feats = word_feats[:, : W_lin.shape[1]]


    loop = jnp.arange(N, dtype=edge_index.dtype)
    srcA = jnp.concatenate([edge_index[0], loop])
    dstA = jnp.concatenate([edge_index[1], loop])
    ewA = jnp.concatenate([edge_attr, jnp.ones((N,), edge_attr.dtype)])
    ET = E + N
    NG = max(16, ((-(-ET // (NW * CHUNK)) + 7) // 8) * 8)
    PW = NG * CHUNK
    EP = NW * PW
    pad = EP - ET
    src = jnp.concatenate([srcA, jnp.zeros((pad,), edge_index.dtype)])
    dst = jnp.concatenate([dstA, jnp.zeros((pad,), edge_index.dtype)])
    ew = jnp.concatenate([ewA, jnp.zeros((pad,), edge_attr.dtype)])
    src3 = src.reshape(NW, NG, CHUNK)
    dst3 = dst.reshape(NW, NG, CHUNK)
    ew3 = ew.reshape(NW, NG, CHUNK)
    edgesP = jnp.stack(
        [src3, dst3, lax.bitcast_convert_type(ew3, jnp.int32)], axis=2
    )

    DEGP = ((N + 2047) // 2048) * 2048
    degp = _make_deg_kernel(NG, PW, DEGP)(dst3, ew).reshape(2, DEGP)
    wout, dinv = _word_dinv(word_feats, W_lin.T, b_lin, degp, N)
    h0 = jnp.concatenate([doc_feats, wout], axis=0)

    FW = NP1 * WP
    H1 = W1.shape[1]
    W1p = jnp.pad(W1, ((0, 0), (0, FW - H1)))
    W1perm = W1p[:, jnp.array(_panel_perm(FW))]
    W1s = jnp.stack([W1perm[:, p * WP:(p + 1) * WP] for p in range(NP1)])
    b1p = jnp.pad(b1, (0, FW - H1))
    b1s = jnp.stack([b1p[p * WP:(p + 1) * WP] for p in range(NP1)]).reshape(
        NP1, 1, WP
    )
    gss = _conv1_mm(h0, W1s, dinv)

    P1 = _make_agg_kernel(N, WP, NP1, NG, True)(gss, edgesP)

    D2 = ((W2.shape[1] + 15) // 16) * 16
    W2p = jnp.pad(W2, ((0, FW - W2.shape[0]), (0, D2 - W2.shape[1])))
    W2s = jnp.stack([W2p[p * WP:(p + 1) * WP] for p in range(NP1)])
    b2p = jnp.pad(b2, (0, D2 - b2.shape[0]))
    gs2 = _conv2_mm(P1, dinv, b1s, W2s)

    P2 = _make_agg_kernel(N, D2, 1, NG, False)(
        gs2.reshape(1, N, D2), edgesP
    )

    out16 = _final(P2, dinv, b2p)
    return out16[:, : W2.shape[1]]


# speedup vs baseline: 1.0728x; 1.0323x over previous
import functools

import jax
import jax.numpy as jnp
from jax import lax
from jax.experimental import pallas as pl
from jax.experimental.pallas import tpu as pltpu
from jax.experimental.pallas import tpu_sc as plsc

NUM_DOCS = 5000
NW = 32
CHUNK = 128
N_TILES = 16
WP = 64
NP1 = 4

_UNPACK_EVEN_ODD = True

_MESH = plsc.VectorSubcoreMesh(core_axis_name="c", subcore_axis_name="s")
_SC_PARAMS = pltpu.CompilerParams(
    needs_layout_passes=False, use_tc_tiling_on_sc=False
)


def _panel_perm(width):
    if not _UNPACK_EVEN_ODD:
        return list(range(width))
    pi = [0] * width
    for j in range(width // 32):
        for m in range(16):
            pi[32 * j + 2 * m] = 32 * j + m
            pi[32 * j + 2 * m + 1] = 32 * j + 16 + m
    return pi


def _linear_body(xr, wr, br, outr):
    outr[...] = (
        jnp.dot(xr[...], wr[...], preferred_element_type=jnp.float32) + br[...]
    )


def _word_linear(word, WlinT, b_lin):
    M, K = word.shape
    Nf = WlinT.shape[1]
    BM = 1000
    return pl.pallas_call(
        _linear_body,
        grid=(M // BM,),
        in_specs=[
            pl.BlockSpec((BM, K), lambda i: (i, 0)),
            pl.BlockSpec((K, Nf), lambda i: (0, 0)),
            pl.BlockSpec((1, Nf), lambda i: (0, 0)),
        ],
        out_specs=pl.BlockSpec((BM, Nf), lambda i: (i, 0)),
        out_shape=jax.ShapeDtypeStruct((M, Nf), jnp.float32),
    )(word, WlinT, b_lin.reshape(1, -1))


def _dinv_body(degr, outr):
    n = outr.shape[0]
    d = degr[0, :n] + degr[1, :n]
    di = jnp.where(d > 0, lax.rsqrt(jnp.where(d > 0, d, 1.0)), 0.0)
    outr[...] = jnp.broadcast_to(di[:, None], outr.shape)


def _dinv_tc(degp, N):
    return pl.pallas_call(
        _dinv_body,
        in_specs=[pl.BlockSpec(degp.shape, lambda: (0, 0))],
        out_specs=pl.BlockSpec((N, 8), lambda: (0, 0)),
        out_shape=jax.ShapeDtypeStruct((N, 8), jnp.float32),
    )(degp)


def _conv1_mm_body(h0r, w1r, dvr, gsr):
    g1 = jnp.dot(h0r[...], w1r[0], preferred_element_type=jnp.float32)
    gsr[0] = (g1 * dvr[:, 0:1]).astype(jnp.bfloat16)


def _conv1_mm(h0, W1s, dinv):
    M, K = h0.shape
    BM = 2000
    return pl.pallas_call(
        _conv1_mm_body,
        grid=(NP1, M // BM),
        in_specs=[
            pl.BlockSpec((BM, K), lambda p, i: (i, 0)),
            pl.BlockSpec((1, K, WP), lambda p, i: (p, 0, 0)),
            pl.BlockSpec((BM, 8), lambda p, i: (i, 0)),
        ],
        out_specs=pl.BlockSpec((1, BM, WP), lambda p, i: (p, i, 0)),
        out_shape=jax.ShapeDtypeStruct((NP1, M, WP), jnp.bfloat16),
    )(h0, W1s, dinv)


def _conv2_mm_body(pr, dvr, b1r, w2r, gs2r):
    dinv = dvr[:, 0:1]
    g2 = jnp.zeros(gs2r.shape, jnp.float32)
    for p in range(NP1):
        agg = pr[0, p] + pr[1, p]
        h1 = jnp.maximum(agg * dinv + b1r[p], 0.0)
        g2 = g2 + jnp.dot(h1, w2r[p], preferred_element_type=jnp.float32)
    gs2r[...] = g2 * dinv


def _conv2_mm(P1, dinv, b1s, W2s):
    M = P1.shape[2]
    D2 = W2s.shape[2]
    BM = 1000
    return pl.pallas_call(
        _conv2_mm_body,
        grid=(M // BM,),
        in_specs=[
            pl.BlockSpec((2, NP1, BM, WP), lambda i: (0, 0, i, 0)),
            pl.BlockSpec((BM, 8), lambda i: (i, 0)),
            pl.BlockSpec((NP1, 1, WP), lambda i: (0, 0, 0)),
            pl.BlockSpec((NP1, WP, D2), lambda i: (0, 0, 0)),
        ],
        out_specs=pl.BlockSpec((BM, D2), lambda i: (i, 0)),
        out_shape=jax.ShapeDtypeStruct((M, D2), jnp.float32),
    )(P1, dinv, b1s, W2s)


def _final_body(pr, dvr, b2r, outr):
    agg = pr[0, 0] + pr[1, 0]
    outr[...] = agg * dvr[:, 0:1] + b2r[...]


def _final(P2, dinv, b2p):
    M, D2 = P2.shape[2], P2.shape[3]
    BM = 1000
    return pl.pallas_call(
        _final_body,
        grid=(M // BM,),
        in_specs=[
            pl.BlockSpec((2, 1, BM, D2), lambda i: (0, 0, i, 0)),
            pl.BlockSpec((BM, 8), lambda i: (i, 0)),
            pl.BlockSpec((1, D2), lambda i: (0, 0)),
        ],
        out_specs=pl.BlockSpec((BM, D2), lambda i: (i, 0)),
        out_shape=jax.ShapeDtypeStruct((M, D2), jnp.float32),
    )(P2, dinv, b2p.reshape(1, -1))


def _make_deg_kernel(NG, PW, DEGP):
    stripe = DEGP // N_TILES
    assert stripe % 128 == 0 and NG % 8 == 0

    @functools.partial(
        pl.kernel,
        out_type=jax.ShapeDtypeStruct((2 * DEGP,), jnp.float32),
        mesh=_MESH,
        compiler_params=_SC_PARAMS,
        scratch_types=[
            pltpu.VMEM_SHARED((DEGP,), jnp.float32),
            pltpu.VMEM((PW,), jnp.float32),
            pltpu.VMEM((NG, CHUNK), jnp.int32),
            pltpu.VMEM((stripe,), jnp.float32),
            pltpu.SemaphoreType.DMA,
        ],
    )
    def deg_kernel(dst_hbm, ew_hbm, out_hbm, acc, ewv, dstv, zv, sem):
        c = lax.axis_index("c")
        s = lax.axis_index("s")
        w = s * 2 + c
        base = pl.multiple_of(s * stripe, 128)

        def zero_body(i, _):
            zv[pl.ds(i * 16, 16)] = jnp.zeros((16,), jnp.float32)
            return 0

        lax.fori_loop(0, stripe // 16, zero_body, 0)
        pltpu.sync_copy(zv, acc.at[pl.ds(base, stripe)])
        plsc.subcore_barrier()

        pltpu.sync_copy(ew_hbm.at[pl.ds(w * PW, PW)], ewv)
        pltpu.sync_copy(dst_hbm.at[w], dstv)

        def scat(g0, _):
            descs = []
            for k in range(8):
                g = g0 * 8 + k
                descs.append(
                    pltpu.async_copy(
                        ewv.at[pl.ds(g * CHUNK, CHUNK)],
                        acc.at[dstv.at[g]],
                        sem,
                        add=True,
                    )
                )
            for d in descs:
                d.wait()
            return 0

        lax.fori_loop(0, NG // 8, scat, 0)
        plsc.subcore_barrier()
        pltpu.sync_copy(
            acc.at[pl.ds(base, stripe)],
            out_hbm.at[pl.ds(pl.multiple_of(c * DEGP + base, 128), stripe)],
        )

    return deg_kernel


def _make_agg_kernel(NROWS, D, NPASS, NG, is_bf16):
    stripe = 624
    last = NROWS - 15 * stripe
    assert NROWS == 10000 and NG % 4 == 0 and NG >= 12
    tdtype = jnp.bfloat16 if is_bf16 else jnp.float32

    @functools.partial(
        pl.kernel,
        out_type=jax.ShapeDtypeStruct((2, NPASS, NROWS, D), jnp.float32),
        mesh=_MESH,
        compiler_params=_SC_PARAMS,
        scratch_types=[
            pltpu.VMEM_SHARED((NROWS, D), jnp.float32),
            pltpu.VMEM_SHARED((NROWS, D), tdtype),
            [pltpu.VMEM((3, CHUNK), jnp.int32) for _ in range(4)],
            [pltpu.VMEM((CHUNK, D), tdtype) for _ in range(4)],
            [pltpu.VMEM((CHUNK, D), jnp.float32) for _ in range(2)],
            [pltpu.VMEM((CHUNK,), jnp.int32) for _ in range(2)],
            [pltpu.SemaphoreType.DMA for _ in range(4)],
            [pltpu.SemaphoreType.DMA for _ in range(4)],
            [pltpu.SemaphoreType.DMA for _ in range(2)],
            pltpu.SemaphoreType.DMA,
            pltpu.SemaphoreType.DMA,
        ],
    )
    def agg_kernel(g_hbm, edges_hbm, out_hbm,
                   acc, tspm, ering, gbufs, sbufs, sidx, esems, gsems, ssems,
                   tsem, osem):
        c = lax.axis_index("c")
        s = lax.axis_index("s")
        w = s * 2 + c

        def row_chunks(emit):
            base = pl.multiple_of(s * stripe, 16)

            @pl.when(s < 15)
            def _():
                for j in range(stripe // CHUNK):
                    emit(pl.multiple_of(base + j * CHUNK, 16), CHUNK)
                r = stripe - (stripe // CHUNK) * CHUNK
                if r:
                    emit(pl.multiple_of(base + stripe - r, 16), r)

            @pl.when(s == 15)
            def _():
                for j in range(last // CHUNK):
                    emit(pl.multiple_of(base + j * CHUNK, 16), CHUNK)
                r = last - (last // CHUNK) * CHUNK
                if r:
                    emit(pl.multiple_of(base + last - r, 16), r)

        def zero_sbuf0(i, _):
            sbufs[0][i // (D // 16), pl.ds((i % (D // 16)) * 16, 16)] = (
                jnp.zeros((16,), jnp.float32)
            )
            return 0

        for p in range(NPASS):
            row_chunks(
                lambda r0, n: pltpu.async_copy(
                    g_hbm.at[p].at[pl.ds(r0, n)], tspm.at[pl.ds(r0, n)], tsem
                )
            )
            if p > 0:
                row_chunks(
                    lambda r0, n: pltpu.make_async_copy(
                        acc.at[pl.ds(r0, n)],
                        out_hbm.at[c, p - 1, pl.ds(r0, n)],
                        osem,
                    ).wait()
                )
            lax.fori_loop(0, CHUNK * (D // 16), zero_sbuf0, 0)
            row_chunks(
                lambda r0, n: pltpu.sync_copy(
                    sbufs[0].at[pl.ds(0, n)], acc.at[pl.ds(r0, n)]
                )
            )
            row_chunks(
                lambda r0, n: pltpu.make_async_copy(
                    g_hbm.at[p].at[pl.ds(r0, n)], tspm.at[pl.ds(r0, n)], tsem
                ).wait()
            )
            plsc.subcore_barrier()
            table = tspm

            def fire_estage(t, k):
                pltpu.async_copy(edges_hbm.at[w].at[t], ering[k], esems[k])

            def wait_estage(t, k):
                pltpu.make_async_copy(
                    edges_hbm.at[w].at[t], ering[k], esems[k]
                ).wait()

            def fire_gather(t, k):
                pltpu.async_copy(table.at[ering[k].at[0]], gbufs[k], gsems[k])

            def wait_gather(t, k):
                pltpu.make_async_copy(
                    table.at[ering[k].at[0]], gbufs[k], gsems[k]
                ).wait()

            def fire_scatter(k):
                pltpu.async_copy(
                    sbufs[k], acc.at[sidx[k]], ssems[k], add=True
                )

            def wait_scatter(k):
                pltpu.make_async_copy(
                    sbufs[k], acc.at[sidx[k]], ssems[k]
                ).wait()

            def compute(t, ke, ks):
                gbuf = gbufs[ke]
                sbuf = sbufs[ks]

                for q in range(CHUNK // 16):
                    sidx[ks][pl.ds(q * 16, 16)] = ering[ke][
                        1, pl.ds(q * 16, 16)
                    ]

                @plsc.parallel_loop(0, CHUNK, unroll=2)
                def body(b):
                    ew = plsc.bitcast(
                        plsc.load_gather(
                            ering[ke],
                            [
                                jnp.full((16,), 2, jnp.int32),
                                jnp.full((16,), b, jnp.int32),
                            ],
                        ),
                        jnp.float32,
                    )
                    if is_bf16:
                        for j in range(D // 32):
                            v = gbuf[b, pl.ds(j * 32, 32)]
                            va, vb = plsc.unpack(
                                v,
                                format=plsc.PackFormat.INTERLEAVED,
                                preferred_element_type=jnp.float32,
                            )
                            sbuf[b, pl.ds(j * 32, 16)] = va * ew
                            sbuf[b, pl.ds(j * 32 + 16, 16)] = vb * ew
                    else:
                        for j in range(D // 16):
                            sbuf[b, pl.ds(j * 16, 16)] = (
                                gbuf[b, pl.ds(j * 16, 16)] * ew
                            )

            def slot(t, ph, first=False, fire_e=True, fire_g=True):
                if fire_e:
                    fire_estage(t + 3, (ph + 3) % 4)
                if fire_g:
                    wait_estage(t + 2, (ph + 2) % 4)
                if not first:
                    wait_scatter(ph % 2)
                if fire_g:
                    fire_gather(t + 2, (ph + 2) % 4)
                wait_gather(t, ph)
                compute(t, ph, ph % 2)
                fire_scatter(ph % 2)

            fire_estage(0, 0)
            fire_estage(1, 1)
            fire_estage(2, 2)
            wait_estage(0, 0)
            fire_gather(0, 0)
            wait_estage(1, 1)
            fire_gather(1, 1)
            slot(0, 0, first=True)
            slot(1, 1, first=True)

            def slot_group(i, _):
                t0 = 2 + i * 4
                for j in range(4):
                    slot(t0 + j, (2 + j) % 4)
                return 0

            lax.fori_loop(0, (NG - 8) // 4, slot_group, 0)

            for t in range(NG - 6, NG):
                slot(t, t % 4, fire_e=(t + 3 < NG), fire_g=(t + 2 < NG))
            wait_scatter((NG - 2) % 2)
            wait_scatter((NG - 1) % 2)
            plsc.subcore_barrier()

            row_chunks(
                lambda r0, n: pltpu.async_copy(
                    acc.at[pl.ds(r0, n)], out_hbm.at[c, p, pl.ds(r0, n)], osem
                )
            )

        lp = NPASS - 1
        row_chunks(
            lambda r0, n: pltpu.make_async_copy(
                acc.at[pl.ds(r0, n)], out_hbm.at[c, lp, pl.ds(r0, n)], osem
            ).wait()
        )

    return agg_kernel


def kernel(x, edge_index, edge_attr, num_docs, W_lin, b_lin, W1, b1, W2, b2):
    N = x.shape[0]
    E = edge_index.shape[1]

    doc_feats = lax.dynamic_slice_in_dim(x, num_docs - NUM_DOCS, NUM_DOCS, axis=0)
    word_feats = lax.dynamic_slice_in_dim(x, num_docs, N - NUM_DOCS, axis=0)
    word_feats = word_feats[:, : W_lin.shape[1]]


    loop = jnp.arange(N, dtype=edge_index.dtype)
    srcA = jnp.concatenate([edge_index[0], loop])
    dstA = jnp.concatenate([edge_index[1], loop])
    ewA = jnp.concatenate([edge_attr, jnp.ones((N,), edge_attr.dtype)])
    ET = E + N
    NG = max(16, ((-(-ET // (NW * CHUNK)) + 7) // 8) * 8)
    PW = NG * CHUNK
    EP = NW * PW
    pad = EP - ET
    src = jnp.concatenate([srcA, jnp.zeros((pad,), edge_index.dtype)])
    dst = jnp.concatenate([dstA, jnp.zeros((pad,), edge_index.dtype)])
    ew = jnp.concatenate([ewA, jnp.zeros((pad,), edge_attr.dtype)])
    src3 = src.reshape(NW, NG, CHUNK)
    dst3 = dst.reshape(NW, NG, CHUNK)
    ew3 = ew.reshape(NW, NG, CHUNK)
    edgesP = jnp.stack(
        [src3, dst3, lax.bitcast_convert_type(ew3, jnp.int32)], axis=2
    )

    DEGP = ((N + 2047) // 2048) * 2048
    degp = _make_deg_kernel(NG, PW, DEGP)(dst3, ew).reshape(2, DEGP)
    wout = _word_linear(word_feats, W_lin.T, b_lin)
    h0 = jnp.concatenate([doc_feats, wout], axis=0)
    dinv = _dinv_tc(degp, N)

    FW = NP1 * WP
    H1 = W1.shape[1]
    W1p = jnp.pad(W1, ((0, 0), (0, FW - H1)))
    W1perm = W1p[:, jnp.array(_panel_perm(FW))]
    W1s = jnp.stack([W1perm[:, p * WP:(p + 1) * WP] for p in range(NP1)])
    b1p = jnp.pad(b1, (0, FW - H1))
    b1s = jnp.stack([b1p[p * WP:(p + 1) * WP] for p in range(NP1)]).reshape(
        NP1, 1, WP
    )
    gss = _conv1_mm(h0, W1s, dinv)

    P1 = _make_agg_kernel(N, WP, NP1, NG, True)(gss, edgesP)

    D2 = ((W2.shape[1] + 15) // 16) * 16
    W2p = jnp.pad(W2, ((0, FW - W2.shape[0]), (0, D2 - W2.shape[1])))
    W2s = jnp.stack([W2p[p * WP:(p + 1) * WP] for p in range(NP1)])
    b2p = jnp.pad(b2, (0, D2 - b2.shape[0]))
    gs2 = _conv2_mm(P1, dinv, b1s, W2s)

    P2 = _make_agg_kernel(N, D2, 1, NG, False)(
        gs2.reshape(1, N, D2), edgesP
    )

    out16 = _final(P2, dinv, b2p)
    return out16[:, : W2.shape[1]]
